# baseline (speedup 1.0000x reference)
import functools

import jax
import jax.numpy as jnp
from jax import lax
from jax.experimental import pallas as pl
from jax.experimental.pallas import tpu as pltpu
from jax.experimental.pallas import tpu_sc as plsc

N_UNITS = 64
IDX_PER_ROW = 128
B_ROWS = 6400
NC = 2
NS = 16
NW = NC * NS
ROWS_PER_W = B_ROWS // NW
CHUNK_ROWS = 8
N_CHUNKS = ROWS_PER_W // CHUNK_ROWS
CHUNK_IDX = CHUNK_ROWS * IDX_PER_ROW
SCALE = 8.0

_mesh = plsc.VectorSubcoreMesh(core_axis_name="c", subcore_axis_name="s")


@functools.partial(
    pl.kernel,
    mesh=_mesh,
    out_type=jax.ShapeDtypeStruct((B_ROWS * IDX_PER_ROW, N_UNITS),
                                  jnp.float32),
    scratch_types=[
        pltpu.VMEM((CHUNK_ROWS, IDX_PER_ROW), jnp.int32),
        pltpu.VMEM((CHUNK_IDX, N_UNITS), jnp.float32),
        pltpu.SemaphoreType.DMA,
    ],
    compiler_params=pltpu.CompilerParams(use_tc_tiling_on_sc=False),
)
def _emb_lookup(x_hbm, table_hbm, out_hbm, idx_v, rows_v, sem):
    wid = lax.axis_index("s") * NC + lax.axis_index("c")
    row_base = wid * ROWS_PER_W

    def chunk(g, carry):
        r0 = row_base + g * CHUNK_ROWS
        pltpu.sync_copy(x_hbm.at[pl.ds(r0, CHUNK_ROWS)], idx_v)
        copies = [
            pltpu.async_copy(
                table_hbm.at[idx_v.at[j]],
                rows_v.at[pl.ds(j * IDX_PER_ROW, IDX_PER_ROW)],
                sem,
            )
            for j in range(CHUNK_ROWS)
        ]
        for cp in copies:
            cp.wait()

        def scale_row(i, c2):
            for o in range(0, N_UNITS, 16):
                rows_v[i, pl.ds(o, 16)] = rows_v[i, pl.ds(o, 16)] * SCALE
            return c2

        lax.fori_loop(0, CHUNK_IDX, scale_row, 0)
        pltpu.sync_copy(rows_v, out_hbm.at[pl.ds(r0 * IDX_PER_ROW, CHUNK_IDX)])
        return carry

    lax.fori_loop(0, N_CHUNKS, chunk, 0)


def kernel(x, table):
    xf = x.reshape(B_ROWS, IDX_PER_ROW)
    out = _emb_lookup(xf, table)
    return out.reshape(x.shape + (N_UNITS,))


# speedup vs baseline: 1.1097x; 1.1097x over previous
import functools

import jax
import jax.numpy as jnp
from jax import lax
from jax.experimental import pallas as pl
from jax.experimental.pallas import tpu as pltpu
from jax.experimental.pallas import tpu_sc as plsc

N_UNITS = 64
CHUNK = 128
B_TOTAL = 4096 * 200
NC = 2
NS = 16
NW = NC * NS
IDX_PER_W = B_TOTAL // NW
N_CHUNKS = IDX_PER_W // CHUNK
NBUF = 8
LOOKAHEAD = 4
SCALE = 8.0

_mesh = plsc.VectorSubcoreMesh(core_axis_name="c", subcore_axis_name="s")


@functools.partial(
    pl.kernel,
    mesh=_mesh,
    out_type=jax.ShapeDtypeStruct((B_TOTAL, N_UNITS), jnp.float32),
    scratch_types=[
        pltpu.VMEM((N_CHUNKS, CHUNK), jnp.int32),
        pltpu.VMEM((NBUF, CHUNK, N_UNITS), jnp.float32),
        pltpu.SemaphoreType.DMA((NBUF,)),
        pltpu.SemaphoreType.DMA((NBUF,)),
    ],
    compiler_params=pltpu.CompilerParams(use_tc_tiling_on_sc=False),
)
def _emb_lookup(x_hbm, table_hbm, out_hbm, idx_v, bufs, gsem, osem):
    wid = lax.axis_index("s") * NC + lax.axis_index("c")
    idx_base = wid * IDX_PER_W

    pltpu.sync_copy(x_hbm.at[pl.ds(wid * N_CHUNKS, N_CHUNKS)], idx_v)

    def fire_gather(chunk_i, b):
        pltpu.async_copy(table_hbm.at[idx_v.at[chunk_i]], bufs.at[b],
                         gsem.at[b])

    def out_slice(chunk_i):
        return out_hbm.at[pl.ds((idx_base + chunk_i * CHUNK), CHUNK)]

    for b in range(LOOKAHEAD):
        fire_gather(b, b)

    def group(h, carry):
        for b in range(NBUF):
            i = h * NBUF + b
            pltpu.make_async_copy(bufs.at[b], out_slice(i), gsem.at[b]).wait()

            def scale(r, c2, _b=b):
                for rr in range(2):
                    for o in range(0, N_UNITS, 16):
                        bufs[_b, r * 2 + rr, pl.ds(o, 16)] = (
                            bufs[_b, r * 2 + rr, pl.ds(o, 16)] * SCALE)
                return c2

            lax.fori_loop(0, CHUNK // 2, scale, 0)

            pltpu.async_copy(bufs.at[b], out_slice(i), osem.at[b])

            bq = (b + LOOKAHEAD) % NBUF

            @pl.when(i >= LOOKAHEAD)
            def _drain():
                pltpu.make_async_copy(bufs.at[bq], out_slice(0),
                                      osem.at[bq]).wait()

            @pl.when(i + LOOKAHEAD < N_CHUNKS)
            def _refill():
                fire_gather(i + LOOKAHEAD, bq)

        return carry

    lax.fori_loop(0, N_CHUNKS // NBUF, group, 0)

    for b in range(LOOKAHEAD, NBUF):
        pltpu.make_async_copy(bufs.at[b], out_slice(0), osem.at[b]).wait()


def kernel(x, table):
    xf = x.reshape(B_TOTAL // CHUNK, CHUNK)
    out = _emb_lookup(xf, table)
    return out.reshape(x.shape + (N_UNITS,))
